# baseline (speedup 1.0000x reference)
import functools

import jax
import jax.numpy as jnp
from jax import lax
from jax.experimental import pallas as pl
from jax.experimental.pallas import tpu as pltpu
from jax.experimental.pallas import tpu_sc as plsc

_ANCHORS = [1.3221, 1.73145, 3.19275, 4.00944, 5.05587, 8.09892, 9.47112,
            4.84053, 11.2364, 10.0071]
_NA = 5
_W = 52
_H = 52
_HW = _H * _W
_G = 16
_NSC = 32
_NTC = 128 - _NSC
_R = _G * _NA


def _prep_kernel(t_ref, p_ref):
    t = t_ref[...]
    gx = t[:, 0:1] * _W
    gy = t[:, 1:2] * _H
    gw = t[:, 2:3] * _W
    gh = t[:, 3:4] * _H
    gif = jnp.floor(gx)
    gjf = jnp.floor(gy)
    garea = gw * gh
    best_iou = jnp.full_like(gx, -1.0)
    best = jnp.zeros_like(gx)
    awb = jnp.zeros_like(gx)
    ahb = jnp.zeros_like(gx)
    for a in range(_NA):
        aw = _ANCHORS[2 * a]
        ah = _ANCHORS[2 * a + 1]
        cw = jnp.minimum(gw, aw)
        ch = jnp.minimum(gh, ah)
        carea = cw * ch
        iou = carea / (garea + aw * ah - carea)
        upd = iou > best_iou
        best = jnp.where(upd, float(a), best)
        awb = jnp.where(upd, aw, awb)
        ahb = jnp.where(upd, ah, ahb)
        best_iou = jnp.where(upd, iou, best_iou)
    tx = gx - gif
    ty = gy - gjf
    tw = jnp.log(gw / awb)
    th = jnp.log(gh / ahb)
    kmatch = gjf * float(_W) + gif
    p_ref[...] = jnp.concatenate(
        [gx, gy, gw, gh, gif, gjf, tx, ty, tw, th, awb, ahb, kmatch, best,
         garea, jnp.zeros_like(gx)], axis=1)


def _dense_kernel(p_ref, x_ref, out_ref):
    g = pl.program_id(0)

    @pl.when(g == 0)
    def _init():
        out_ref[0:1, 0:1] = jnp.zeros((1, 1), jnp.float32)

    x4 = x_ref[...].reshape(_R, _NA, _HW)
    X = x4[:, 0, :]
    Y = x4[:, 1, :]
    Wc = x4[:, 2, :]
    Hc = x4[:, 3, :]
    C = x4[:, 4, :]

    rio = jax.lax.broadcasted_iota(jnp.int32, (_R, _G), 0) // _NA
    cio = jax.lax.broadcasted_iota(jnp.int32, (_R, _G), 1)
    E = (rio == cio).astype(jnp.float32)
    EP = jnp.dot(E, p_ref[...], preferred_element_type=jnp.float32)

    def col(i):
        return EP[:, i:i + 1]

    gx, gy, gw, gh = col(0), col(1), col(2), col(3)
    gif, gjf = col(4), col(5)
    tx, ty, tw, th = col(6), col(7), col(8), col(9)
    kmatch, best, garea = col(12), col(13), col(14)

    aidx = (jax.lax.broadcasted_iota(jnp.int32, (_R, 1), 0) % _NA
            ).astype(jnp.float32)
    anw = jnp.zeros((_R, 1), jnp.float32)
    anh = jnp.zeros((_R, 1), jnp.float32)
    for a in range(_NA):
        m = aidx == float(a)
        anw = jnp.where(m, _ANCHORS[2 * a], anw)
        anh = jnp.where(m, _ANCHORS[2 * a + 1], anh)

    kiof = jax.lax.broadcasted_iota(
        jnp.int32, (_R, _HW), 1).astype(jnp.float32)
    gridy = jnp.floor(kiof * (1.0 / _W))
    gridx = kiof - gridy * _W

    gx0 = gx - 0.5 * gw
    gx1 = gx + 0.5 * gw
    gy0 = gy - 0.5 * gh
    gy1 = gy + 0.5 * gh

    dx = 0.5 * jnp.tanh(0.5 * X)
    dy = 0.5 * jnp.tanh(0.5 * Y)
    pc = 0.5 + 0.5 * jnp.tanh(0.5 * C)
    pwv = jnp.exp(Wc) * anw
    phv = jnp.exp(Hc) * anh
    pxv = dx + (gridx + 0.5)
    pyv = dy + (gridy + 0.5)
    hw_ = 0.5 * pwv
    hh_ = 0.5 * phv
    uw = jnp.maximum(gx1, pxv + hw_) - jnp.minimum(gx0, pxv - hw_)
    uh = jnp.maximum(gy1, pyv + hh_) - jnp.minimum(gy0, pyv - hh_)
    cw = gw + pwv - uw
    ch = gh + phv - uh
    carea = cw * ch
    uarea = garea + pwv * phv - carea
    sil = (cw > 0.0) & (ch > 0.0) & (carea > 0.6 * uarea)
    cell = dx * dx + dy * dy + Wc * Wc + Hc * Hc \
        + jnp.where(sil, 0.0, pc * pc)
    base = jnp.sum(cell, axis=1, keepdims=True)

    rowm = best == aidx
    sel = rowm & (kiof == kmatch)
    r0 = jnp.sum(jnp.where(sel, X, 0.0), axis=1, keepdims=True)
    r1 = jnp.sum(jnp.where(sel, Y, 0.0), axis=1, keepdims=True)
    r2 = jnp.sum(jnp.where(sel, Wc, 0.0), axis=1, keepdims=True)
    r3 = jnp.sum(jnp.where(sel, Hc, 0.0), axis=1, keepdims=True)
    r4 = jnp.sum(jnp.where(sel, C, 0.0), axis=1, keepdims=True)

    sxm = 0.5 + 0.5 * jnp.tanh(0.5 * r0)
    sym = 0.5 + 0.5 * jnp.tanh(0.5 * r1)
    pcm = 0.5 + 0.5 * jnp.tanh(0.5 * r4)
    pwm = jnp.exp(r2) * anw
    phm = jnp.exp(r3) * anh
    pxm = sxm + gif
    pym = sym + gjf
    uwm = jnp.maximum(gx1, pxm + 0.5 * pwm) - jnp.minimum(gx0, pxm - 0.5 * pwm)
    uhm = jnp.maximum(gy1, pym + 0.5 * phm) - jnp.minimum(gy0, pym - 0.5 * phm)
    cwm = gw + pwm - uwm
    chm = gh + phm - uhm
    cam = cwm * chm
    uam = garea + pwm * phm - cam
    iou_t = jnp.where((cwm > 0.0) & (chm > 0.0), cam / uam, 0.0)

    coord_corr = (sxm - tx) ** 2 - (sxm - 0.5) ** 2 \
        + (sym - ty) ** 2 - (sym - 0.5) ** 2 \
        + (r2 - tw) ** 2 - r2 * r2 \
        + (r3 - th) ** 2 - r3 * r3
    dconf = pcm - iou_t
    conf_corr = 5.0 * dconf * dconf \
        - jnp.where(iou_t > 0.6, 0.0, pcm * pcm)
    corr = jnp.where(rowm, coord_corr + conf_corr, 0.0)

    step = jnp.sum(base + corr, axis=0, keepdims=True)
    out_ref[0:1, 0:1] += step[0:1, 0:1]


def _sc_kernel(pred_hbm, params_hbm, out_hbm, chunk_v, parm_v, acc_v):
    nc = 2
    wid = lax.axis_index("s") * nc + lax.axis_index("c")
    sample = _NTC + wid

    pltpu.sync_copy(params_hbm.at[pl.ds(sample * 256, 256)], parm_v)

    def bcast(j):
        return parm_v[pl.ds(j * 16, 16)]

    gx, gy, gw, gh = bcast(0), bcast(1), bcast(2), bcast(3)
    gif, gjf = bcast(4), bcast(5)
    tx, ty, tw, th = bcast(6), bcast(7), bcast(8), bcast(9)
    awb, ahb = bcast(10), bcast(11)
    kmatch, best, garea = bcast(12), bcast(13), bcast(14)

    gx0 = gx - 0.5 * gw
    gx1 = gx + 0.5 * gw
    gy0 = gy - 0.5 * gh
    gy1 = gy + 0.5 * gh

    lane = lax.iota(jnp.int32, 16)
    zero = jnp.zeros((16,), jnp.float32)

    acc = zero
    raws = [zero] * 5
    for a in range(_NA):
        pltpu.sync_copy(
            pred_hbm.at[pl.ds((sample * 25 + _NA * a) * _HW, _NA * _HW)],
            chunk_v)
        anw = float(_ANCHORS[2 * a])
        anh = float(_ANCHORS[2 * a + 1])
        besteq = best == float(a)

        def body(i, carry):
            acc, rx, ry, rw, rh, rc = carry
            k = i * 16 + lane
            kf = k.astype(jnp.float32)
            gyi = ((k * 40330) >> 21).astype(jnp.float32)
            gxi = kf - gyi * _W
            xr = chunk_v[pl.ds(0 * _HW + i * 16, 16)]
            yr = chunk_v[pl.ds(1 * _HW + i * 16, 16)]
            wr = chunk_v[pl.ds(2 * _HW + i * 16, 16)]
            hr = chunk_v[pl.ds(3 * _HW + i * 16, 16)]
            cr = chunk_v[pl.ds(4 * _HW + i * 16, 16)]
            sigx = 1.0 / (1.0 + jnp.exp(-xr))
            sigy = 1.0 / (1.0 + jnp.exp(-yr))
            pcv = 1.0 / (1.0 + jnp.exp(-cr))
            pwv = jnp.exp(wr) * anw
            phv = jnp.exp(hr) * anh
            pxv = sigx + gxi
            pyv = sigy + gyi
            hw_ = 0.5 * pwv
            hh_ = 0.5 * phv
            uw = jnp.maximum(gx1, pxv + hw_) - jnp.minimum(gx0, pxv - hw_)
            uh = jnp.maximum(gy1, pyv + hh_) - jnp.minimum(gy0, pyv - hh_)
            cwv = gw + pwv - uw
            chv = gh + phv - uh
            carea = cwv * chv
            uarea = garea + pwv * phv - carea
            sil = (cwv > 0.0) & (chv > 0.0) & (carea > 0.6 * uarea)
            ddx = sigx - 0.5
            ddy = sigy - 0.5
            cell = ddx * ddx + ddy * ddy + wr * wr + hr * hr \
                + jnp.where(sil, 0.0, pcv * pcv)
            selm = besteq & (kf == kmatch)
            rx = rx + jnp.where(selm, xr, 0.0)
            ry = ry + jnp.where(selm, yr, 0.0)
            rw = rw + jnp.where(selm, wr, 0.0)
            rh = rh + jnp.where(selm, hr, 0.0)
            rc = rc + jnp.where(selm, cr, 0.0)
            return (acc + cell, rx, ry, rw, rh, rc)

        acc, raws[0], raws[1], raws[2], raws[3], raws[4] = lax.fori_loop(
            0, _HW // 16, body, (acc, raws[0], raws[1], raws[2], raws[3],
                                 raws[4]))

    def tot(v):
        for bit in (1, 2, 4, 8):
            idx = lane ^ bit
            v = v + v.at[idx].get(mode="promise_in_bounds")
        return v

    r0, r1, r2, r3, r4 = (tot(v) for v in raws)
    sxm = 1.0 / (1.0 + jnp.exp(-r0))
    sym = 1.0 / (1.0 + jnp.exp(-r1))
    pcm = 1.0 / (1.0 + jnp.exp(-r4))
    pwm = jnp.exp(r2) * awb
    phm = jnp.exp(r3) * ahb
    pxm = sxm + gif
    pym = sym + gjf
    uwm = jnp.maximum(gx1, pxm + 0.5 * pwm) - jnp.minimum(gx0, pxm - 0.5 * pwm)
    uhm = jnp.maximum(gy1, pym + 0.5 * phm) - jnp.minimum(gy0, pym - 0.5 * phm)
    cwm = gw + pwm - uwm
    chm = gh + phm - uhm
    cam = cwm * chm
    uam = garea + pwm * phm - cam
    iou_t = jnp.where((cwm > 0.0) & (chm > 0.0), cam / uam, 0.0)

    def sq(v):
        return v * v

    coord_corr = sq(sxm - tx) - sq(sxm - 0.5) \
        + sq(sym - ty) - sq(sym - 0.5) \
        + sq(r2 - tw) - r2 * r2 \
        + sq(r3 - th) - r3 * r3
    dconf = pcm - iou_t
    conf_corr = 5.0 * dconf * dconf \
        - jnp.where(iou_t > 0.6, 0.0, pcm * pcm)
    acc = acc + (coord_corr + conf_corr) * (1.0 / 16.0)

    acc_v[...] = acc
    pltpu.sync_copy(acc_v, out_hbm.at[pl.ds(wid * 16, 16)])


def kernel(pred, target):
    bs = pred.shape[0]
    pred3 = pred.reshape(bs, _NA * 5, _HW)
    params = pl.pallas_call(
        _prep_kernel,
        out_shape=jax.ShapeDtypeStruct((bs, 16), jnp.float32),
    )(target)

    tc_total = pl.pallas_call(
        _dense_kernel,
        grid=(_NTC // _G,),
        in_specs=[
            pl.BlockSpec((_G, 16), lambda g: (g, 0)),
            pl.BlockSpec((_G, _NA * 5, _HW), lambda g: (g, 0, 0)),
        ],
        out_specs=pl.BlockSpec((1, 1), lambda g: (0, 0)),
        out_shape=jax.ShapeDtypeStruct((1, 1), jnp.float32),
    )(params[:_NTC], pred3[:_NTC])

    mesh = plsc.VectorSubcoreMesh(core_axis_name="c", subcore_axis_name="s")
    sc_part = functools.partial(
        pl.kernel, mesh=mesh,
        out_type=jax.ShapeDtypeStruct((_NSC * 16,), jnp.float32),
        scratch_types=[
            pltpu.VMEM((_NA * _HW,), jnp.float32),
            pltpu.VMEM((256,), jnp.float32),
            pltpu.VMEM((16,), jnp.float32),
        ],
    )(_sc_kernel)
    params_sc = jnp.broadcast_to(
        params[:, :, None], (bs, 16, 16)).reshape(-1)
    sc_out = sc_part(pred3.reshape(-1), params_sc)

    return (tc_total[0, 0] + jnp.sum(sc_out)) * 0.5


# speedup vs baseline: 1.1639x; 1.1639x over previous
import functools

import jax
import jax.numpy as jnp
from jax import lax
from jax.experimental import pallas as pl
from jax.experimental.pallas import tpu as pltpu
from jax.experimental.pallas import tpu_sc as plsc

_ANCHORS = [1.3221, 1.73145, 3.19275, 4.00944, 5.05587, 8.09892, 9.47112,
            4.84053, 11.2364, 10.0071]
_NA = 5
_W = 52
_H = 52
_HW = _H * _W
_G = 16
_NSC = 32
_NTC = 128 - _NSC
_R = _G * _NA


def _prep_kernel(t_ref, p_ref):
    t = t_ref[...]
    gx = t[:, 0:1] * _W
    gy = t[:, 1:2] * _H
    gw = t[:, 2:3] * _W
    gh = t[:, 3:4] * _H
    gif = jnp.floor(gx)
    gjf = jnp.floor(gy)
    garea = gw * gh
    best_iou = jnp.full_like(gx, -1.0)
    best = jnp.zeros_like(gx)
    awb = jnp.zeros_like(gx)
    ahb = jnp.zeros_like(gx)
    for a in range(_NA):
        aw = _ANCHORS[2 * a]
        ah = _ANCHORS[2 * a + 1]
        cw = jnp.minimum(gw, aw)
        ch = jnp.minimum(gh, ah)
        carea = cw * ch
        iou = carea / (garea + aw * ah - carea)
        upd = iou > best_iou
        best = jnp.where(upd, float(a), best)
        awb = jnp.where(upd, aw, awb)
        ahb = jnp.where(upd, ah, ahb)
        best_iou = jnp.where(upd, iou, best_iou)
    tx = gx - gif
    ty = gy - gjf
    tw = jnp.log(gw / awb)
    th = jnp.log(gh / ahb)
    kmatch = gjf * float(_W) + gif
    p_ref[...] = jnp.concatenate(
        [gx, gy, gw, gh, gif, gjf, tx, ty, tw, th, awb, ahb, kmatch, best,
         garea, jnp.zeros_like(gx)], axis=1)


def _dense_kernel(p_ref, x_ref, out_ref):
    g = pl.program_id(0)

    @pl.when(g == 0)
    def _init():
        out_ref[0:1, 0:1] = jnp.zeros((1, 1), jnp.float32)

    x4 = x_ref[...].reshape(_R, _NA, _HW)
    X = x4[:, 0, :]
    Y = x4[:, 1, :]
    Wc = x4[:, 2, :]
    Hc = x4[:, 3, :]
    C = x4[:, 4, :]

    rio = jax.lax.broadcasted_iota(jnp.int32, (_R, _G), 0) // _NA
    cio = jax.lax.broadcasted_iota(jnp.int32, (_R, _G), 1)
    E = (rio == cio).astype(jnp.float32)
    EP = jnp.dot(E, p_ref[...], preferred_element_type=jnp.float32)

    def col(i):
        return EP[:, i:i + 1]

    gx, gy, gw, gh = col(0), col(1), col(2), col(3)
    gif, gjf = col(4), col(5)
    tx, ty, tw, th = col(6), col(7), col(8), col(9)
    kmatch, best, garea = col(12), col(13), col(14)

    aidx = (jax.lax.broadcasted_iota(jnp.int32, (_R, 1), 0) % _NA
            ).astype(jnp.float32)
    anw = jnp.zeros((_R, 1), jnp.float32)
    anh = jnp.zeros((_R, 1), jnp.float32)
    for a in range(_NA):
        m = aidx == float(a)
        anw = jnp.where(m, _ANCHORS[2 * a], anw)
        anh = jnp.where(m, _ANCHORS[2 * a + 1], anh)

    kiof = jax.lax.broadcasted_iota(
        jnp.int32, (_R, _HW), 1).astype(jnp.float32)
    gridy = jnp.floor(kiof * (1.0 / _W))
    gridx = kiof - gridy * _W

    gx0 = gx - 0.5 * gw
    gx1 = gx + 0.5 * gw
    gy0 = gy - 0.5 * gh
    gy1 = gy + 0.5 * gh

    dx = 0.5 * jnp.tanh(0.5 * X)
    dy = 0.5 * jnp.tanh(0.5 * Y)
    pc = 0.5 + 0.5 * jnp.tanh(0.5 * C)
    pwv = jnp.exp(Wc) * anw
    phv = jnp.exp(Hc) * anh
    pxv = dx + (gridx + 0.5)
    pyv = dy + (gridy + 0.5)
    hw_ = 0.5 * pwv
    hh_ = 0.5 * phv
    uw = jnp.maximum(gx1, pxv + hw_) - jnp.minimum(gx0, pxv - hw_)
    uh = jnp.maximum(gy1, pyv + hh_) - jnp.minimum(gy0, pyv - hh_)
    cw = gw + pwv - uw
    ch = gh + phv - uh
    carea = cw * ch
    uarea = garea + pwv * phv - carea
    sil = (cw > 0.0) & (ch > 0.0) & (carea > 0.6 * uarea)
    cell = dx * dx + dy * dy + Wc * Wc + Hc * Hc \
        + jnp.where(sil, 0.0, pc * pc)
    base = jnp.sum(cell, axis=1, keepdims=True)

    rowm = best == aidx
    sel = rowm & (kiof == kmatch)
    r0 = jnp.sum(jnp.where(sel, X, 0.0), axis=1, keepdims=True)
    r1 = jnp.sum(jnp.where(sel, Y, 0.0), axis=1, keepdims=True)
    r2 = jnp.sum(jnp.where(sel, Wc, 0.0), axis=1, keepdims=True)
    r3 = jnp.sum(jnp.where(sel, Hc, 0.0), axis=1, keepdims=True)
    r4 = jnp.sum(jnp.where(sel, C, 0.0), axis=1, keepdims=True)

    sxm = 0.5 + 0.5 * jnp.tanh(0.5 * r0)
    sym = 0.5 + 0.5 * jnp.tanh(0.5 * r1)
    pcm = 0.5 + 0.5 * jnp.tanh(0.5 * r4)
    pwm = jnp.exp(r2) * anw
    phm = jnp.exp(r3) * anh
    pxm = sxm + gif
    pym = sym + gjf
    uwm = jnp.maximum(gx1, pxm + 0.5 * pwm) - jnp.minimum(gx0, pxm - 0.5 * pwm)
    uhm = jnp.maximum(gy1, pym + 0.5 * phm) - jnp.minimum(gy0, pym - 0.5 * phm)
    cwm = gw + pwm - uwm
    chm = gh + phm - uhm
    cam = cwm * chm
    uam = garea + pwm * phm - cam
    iou_t = jnp.where((cwm > 0.0) & (chm > 0.0), cam / uam, 0.0)

    coord_corr = (sxm - tx) ** 2 - (sxm - 0.5) ** 2 \
        + (sym - ty) ** 2 - (sym - 0.5) ** 2 \
        + (r2 - tw) ** 2 - r2 * r2 \
        + (r3 - th) ** 2 - r3 * r3
    dconf = pcm - iou_t
    conf_corr = 5.0 * dconf * dconf \
        - jnp.where(iou_t > 0.6, 0.0, pcm * pcm)
    corr = jnp.where(rowm, coord_corr + conf_corr, 0.0)

    step = jnp.sum(base + corr, axis=0, keepdims=True)
    out_ref[0:1, 0:1] += step[0:1, 0:1]


def _sc_kernel(pred_hbm, params_hbm, out_hbm, chunk_v, parm_v, acc_v):
    nc = 2
    wid = lax.axis_index("s") * nc + lax.axis_index("c")
    sample = _NTC + wid

    pltpu.sync_copy(params_hbm.at[pl.ds(sample * 256, 256)], parm_v)

    def bcast(j):
        return parm_v[pl.ds(j * 16, 16)]

    gx, gy, gw, gh = bcast(0), bcast(1), bcast(2), bcast(3)
    gif, gjf = bcast(4), bcast(5)
    tx, ty, tw, th = bcast(6), bcast(7), bcast(8), bcast(9)
    awb, ahb = bcast(10), bcast(11)
    kmatch, best, garea = bcast(12), bcast(13), bcast(14)

    gx0 = gx - 0.5 * gw
    gx1 = gx + 0.5 * gw
    gy0 = gy - 0.5 * gh
    gy1 = gy + 0.5 * gh

    lane = lax.iota(jnp.int32, 16)
    zero = jnp.zeros((16,), jnp.float32)

    acc = zero
    raws = [zero] * 5
    pltpu.sync_copy(pred_hbm.at[sample], chunk_v)
    for a in range(_NA):
        anw = float(_ANCHORS[2 * a])
        anh = float(_ANCHORS[2 * a + 1])
        besteq = best == float(a)

        def body(i, carry):
            acc, rx, ry, rw, rh, rc = carry
            k = i * 16 + lane
            kf = k.astype(jnp.float32)
            gyi = ((k * 40330) >> 21).astype(jnp.float32)
            gxi = kf - gyi * _W
            xr = chunk_v[5 * a + 0, pl.ds(i * 16, 16)]
            yr = chunk_v[5 * a + 1, pl.ds(i * 16, 16)]
            wr = chunk_v[5 * a + 2, pl.ds(i * 16, 16)]
            hr = chunk_v[5 * a + 3, pl.ds(i * 16, 16)]
            cr = chunk_v[5 * a + 4, pl.ds(i * 16, 16)]
            sigx = 1.0 / (1.0 + jnp.exp(-xr))
            sigy = 1.0 / (1.0 + jnp.exp(-yr))
            pcv = 1.0 / (1.0 + jnp.exp(-cr))
            pwv = jnp.exp(wr) * anw
            phv = jnp.exp(hr) * anh
            pxv = sigx + gxi
            pyv = sigy + gyi
            hw_ = 0.5 * pwv
            hh_ = 0.5 * phv
            uw = jnp.maximum(gx1, pxv + hw_) - jnp.minimum(gx0, pxv - hw_)
            uh = jnp.maximum(gy1, pyv + hh_) - jnp.minimum(gy0, pyv - hh_)
            cwv = gw + pwv - uw
            chv = gh + phv - uh
            carea = cwv * chv
            uarea = garea + pwv * phv - carea
            sil = (cwv > 0.0) & (chv > 0.0) & (carea > 0.6 * uarea)
            ddx = sigx - 0.5
            ddy = sigy - 0.5
            cell = ddx * ddx + ddy * ddy + wr * wr + hr * hr \
                + jnp.where(sil, 0.0, pcv * pcv)
            selm = besteq & (kf == kmatch)
            rx = rx + jnp.where(selm, xr, 0.0)
            ry = ry + jnp.where(selm, yr, 0.0)
            rw = rw + jnp.where(selm, wr, 0.0)
            rh = rh + jnp.where(selm, hr, 0.0)
            rc = rc + jnp.where(selm, cr, 0.0)
            return (acc + cell, rx, ry, rw, rh, rc)

        acc, raws[0], raws[1], raws[2], raws[3], raws[4] = lax.fori_loop(
            0, _HW // 16, body, (acc, raws[0], raws[1], raws[2], raws[3],
                                 raws[4]))

    def tot(v):
        for bit in (1, 2, 4, 8):
            idx = lane ^ bit
            v = v + v.at[idx].get(mode="promise_in_bounds")
        return v

    r0, r1, r2, r3, r4 = (tot(v) for v in raws)
    sxm = 1.0 / (1.0 + jnp.exp(-r0))
    sym = 1.0 / (1.0 + jnp.exp(-r1))
    pcm = 1.0 / (1.0 + jnp.exp(-r4))
    pwm = jnp.exp(r2) * awb
    phm = jnp.exp(r3) * ahb
    pxm = sxm + gif
    pym = sym + gjf
    uwm = jnp.maximum(gx1, pxm + 0.5 * pwm) - jnp.minimum(gx0, pxm - 0.5 * pwm)
    uhm = jnp.maximum(gy1, pym + 0.5 * phm) - jnp.minimum(gy0, pym - 0.5 * phm)
    cwm = gw + pwm - uwm
    chm = gh + phm - uhm
    cam = cwm * chm
    uam = garea + pwm * phm - cam
    iou_t = jnp.where((cwm > 0.0) & (chm > 0.0), cam / uam, 0.0)

    def sq(v):
        return v * v

    coord_corr = sq(sxm - tx) - sq(sxm - 0.5) \
        + sq(sym - ty) - sq(sym - 0.5) \
        + sq(r2 - tw) - r2 * r2 \
        + sq(r3 - th) - r3 * r3
    dconf = pcm - iou_t
    conf_corr = 5.0 * dconf * dconf \
        - jnp.where(iou_t > 0.6, 0.0, pcm * pcm)
    acc = acc + (coord_corr + conf_corr) * (1.0 / 16.0)

    acc_v[...] = acc
    pltpu.sync_copy(acc_v, out_hbm.at[pl.ds(wid * 16, 16)])


def kernel(pred, target):
    bs = pred.shape[0]
    pred3 = pred.reshape(bs, _NA * 5, _HW)
    params = pl.pallas_call(
        _prep_kernel,
        out_shape=jax.ShapeDtypeStruct((bs, 16), jnp.float32),
    )(target)

    tc_total = pl.pallas_call(
        _dense_kernel,
        grid=(_NTC // _G,),
        in_specs=[
            pl.BlockSpec((_G, 16), lambda g: (g, 0)),
            pl.BlockSpec((_G, _NA * 5, _HW), lambda g: (g, 0, 0)),
        ],
        out_specs=pl.BlockSpec((1, 1), lambda g: (0, 0)),
        out_shape=jax.ShapeDtypeStruct((1, 1), jnp.float32),
    )(params[:_NTC], pred3[:_NTC])

    mesh = plsc.VectorSubcoreMesh(core_axis_name="c", subcore_axis_name="s")
    sc_part = functools.partial(
        pl.kernel, mesh=mesh,
        out_type=jax.ShapeDtypeStruct((_NSC * 16,), jnp.float32),
        scratch_types=[
            pltpu.VMEM((_NA * 5, _HW), jnp.float32),
            pltpu.VMEM((256,), jnp.float32),
            pltpu.VMEM((16,), jnp.float32),
        ],
    )(_sc_kernel)
    params_sc = jnp.broadcast_to(
        params[:, :, None], (bs, 16, 16)).reshape(-1)
    sc_out = sc_part(pred3, params_sc)

    return (tc_total[0, 0] + jnp.sum(sc_out)) * 0.5


# speedup vs baseline: 1.8063x; 1.5520x over previous
import functools

import jax
import jax.numpy as jnp
from jax import lax
from jax.experimental import pallas as pl
from jax.experimental.pallas import tpu as pltpu
from jax.experimental.pallas import tpu_sc as plsc

_ANCHORS = [1.3221, 1.73145, 3.19275, 4.00944, 5.05587, 8.09892, 9.47112,
            4.84053, 11.2364, 10.0071]
_NA = 5
_W = 52
_H = 52
_HW = _H * _W
_G = 16
_NSC = 32
_NTC = 128 - _NSC
_R = _G * _NA


def _prep_kernel(t_ref, p_ref):
    t = t_ref[...]
    gx = t[:, 0:1] * _W
    gy = t[:, 1:2] * _H
    gw = t[:, 2:3] * _W
    gh = t[:, 3:4] * _H
    gif = jnp.floor(gx)
    gjf = jnp.floor(gy)
    garea = gw * gh
    best_iou = jnp.full_like(gx, -1.0)
    best = jnp.zeros_like(gx)
    awb = jnp.zeros_like(gx)
    ahb = jnp.zeros_like(gx)
    for a in range(_NA):
        aw = _ANCHORS[2 * a]
        ah = _ANCHORS[2 * a + 1]
        cw = jnp.minimum(gw, aw)
        ch = jnp.minimum(gh, ah)
        carea = cw * ch
        iou = carea / (garea + aw * ah - carea)
        upd = iou > best_iou
        best = jnp.where(upd, float(a), best)
        awb = jnp.where(upd, aw, awb)
        ahb = jnp.where(upd, ah, ahb)
        best_iou = jnp.where(upd, iou, best_iou)
    tx = gx - gif
    ty = gy - gjf
    tw = jnp.log(gw / awb)
    th = jnp.log(gh / ahb)
    kmatch = gjf * float(_W) + gif
    p_ref[...] = jnp.concatenate(
        [gx, gy, gw, gh, gif, gjf, tx, ty, tw, th, awb, ahb, kmatch, best,
         garea, jnp.zeros_like(gx)], axis=1)


def _dense_kernel(p_ref, x_ref, out_ref):
    g = pl.program_id(0)

    @pl.when(g == 0)
    def _init():
        out_ref[0:1, 0:1] = jnp.zeros((1, 1), jnp.float32)

    x4 = x_ref[...].reshape(_R, _NA, _HW)
    X = x4[:, 0, :]
    Y = x4[:, 1, :]
    Wc = x4[:, 2, :]
    Hc = x4[:, 3, :]
    C = x4[:, 4, :]

    rio = jax.lax.broadcasted_iota(jnp.int32, (_R, _G), 0) // _NA
    cio = jax.lax.broadcasted_iota(jnp.int32, (_R, _G), 1)
    E = (rio == cio).astype(jnp.float32)
    EP = jnp.dot(E, p_ref[...], preferred_element_type=jnp.float32)

    def col(i):
        return EP[:, i:i + 1]

    gx, gy, gw, gh = col(0), col(1), col(2), col(3)
    gif, gjf = col(4), col(5)
    tx, ty, tw, th = col(6), col(7), col(8), col(9)
    kmatch, best, garea = col(12), col(13), col(14)

    aidx = (jax.lax.broadcasted_iota(jnp.int32, (_R, 1), 0) % _NA
            ).astype(jnp.float32)
    anw = jnp.zeros((_R, 1), jnp.float32)
    anh = jnp.zeros((_R, 1), jnp.float32)
    for a in range(_NA):
        m = aidx == float(a)
        anw = jnp.where(m, _ANCHORS[2 * a], anw)
        anh = jnp.where(m, _ANCHORS[2 * a + 1], anh)

    kiof = jax.lax.broadcasted_iota(
        jnp.int32, (_R, _HW), 1).astype(jnp.float32)
    gridy = jnp.floor(kiof * (1.0 / _W))
    gridx = kiof - gridy * _W

    gx0 = gx - 0.5 * gw
    gx1 = gx + 0.5 * gw
    gy0 = gy - 0.5 * gh
    gy1 = gy + 0.5 * gh

    dx = 0.5 * jnp.tanh(0.5 * X)
    dy = 0.5 * jnp.tanh(0.5 * Y)
    pc = 0.5 + 0.5 * jnp.tanh(0.5 * C)
    pwv = jnp.exp(Wc) * anw
    phv = jnp.exp(Hc) * anh
    pxv = dx + (gridx + 0.5)
    pyv = dy + (gridy + 0.5)
    hw_ = 0.5 * pwv
    hh_ = 0.5 * phv
    uw = jnp.maximum(gx1, pxv + hw_) - jnp.minimum(gx0, pxv - hw_)
    uh = jnp.maximum(gy1, pyv + hh_) - jnp.minimum(gy0, pyv - hh_)
    cw = gw + pwv - uw
    ch = gh + phv - uh
    carea = cw * ch
    uarea = garea + pwv * phv - carea
    sil = (cw > 0.0) & (ch > 0.0) & (carea > 0.6 * uarea)
    cell = dx * dx + dy * dy + Wc * Wc + Hc * Hc \
        + jnp.where(sil, 0.0, pc * pc)
    base = jnp.sum(cell, axis=1, keepdims=True)

    rowm = best == aidx
    sel = rowm & (kiof == kmatch)
    r0 = jnp.sum(jnp.where(sel, X, 0.0), axis=1, keepdims=True)
    r1 = jnp.sum(jnp.where(sel, Y, 0.0), axis=1, keepdims=True)
    r2 = jnp.sum(jnp.where(sel, Wc, 0.0), axis=1, keepdims=True)
    r3 = jnp.sum(jnp.where(sel, Hc, 0.0), axis=1, keepdims=True)
    r4 = jnp.sum(jnp.where(sel, C, 0.0), axis=1, keepdims=True)

    sxm = 0.5 + 0.5 * jnp.tanh(0.5 * r0)
    sym = 0.5 + 0.5 * jnp.tanh(0.5 * r1)
    pcm = 0.5 + 0.5 * jnp.tanh(0.5 * r4)
    pwm = jnp.exp(r2) * anw
    phm = jnp.exp(r3) * anh
    pxm = sxm + gif
    pym = sym + gjf
    uwm = jnp.maximum(gx1, pxm + 0.5 * pwm) - jnp.minimum(gx0, pxm - 0.5 * pwm)
    uhm = jnp.maximum(gy1, pym + 0.5 * phm) - jnp.minimum(gy0, pym - 0.5 * phm)
    cwm = gw + pwm - uwm
    chm = gh + phm - uhm
    cam = cwm * chm
    uam = garea + pwm * phm - cam
    iou_t = jnp.where((cwm > 0.0) & (chm > 0.0), cam / uam, 0.0)

    coord_corr = (sxm - tx) ** 2 - (sxm - 0.5) ** 2 \
        + (sym - ty) ** 2 - (sym - 0.5) ** 2 \
        + (r2 - tw) ** 2 - r2 * r2 \
        + (r3 - th) ** 2 - r3 * r3
    dconf = pcm - iou_t
    conf_corr = 5.0 * dconf * dconf \
        - jnp.where(iou_t > 0.6, 0.0, pcm * pcm)
    corr = jnp.where(rowm, coord_corr + conf_corr, 0.0)

    step = jnp.sum(base + corr, axis=0, keepdims=True)
    out_ref[0:1, 0:1] += step[0:1, 0:1]


def _sc_kernel(pred_hbm, params_hbm, out_hbm, chunk_v, parm_v, acc_v):
    nc = 2
    wid = lax.axis_index("s") * nc + lax.axis_index("c")
    sample = _NTC + wid

    pltpu.sync_copy(params_hbm.at[pl.ds(sample * 256, 256)], parm_v)

    def bcast(j):
        return parm_v[pl.ds(j * 16, 16)]

    gx, gy, gw, gh = bcast(0), bcast(1), bcast(2), bcast(3)
    gif, gjf = bcast(4), bcast(5)
    tx, ty, tw, th = bcast(6), bcast(7), bcast(8), bcast(9)
    awb, ahb = bcast(10), bcast(11)
    kmatch, best, garea = bcast(12), bcast(13), bcast(14)

    gx0 = gx - 0.5 * gw
    gx1 = gx + 0.5 * gw
    gy0 = gy - 0.5 * gh
    gy1 = gy + 0.5 * gh

    lane = lax.iota(jnp.int32, 16)
    zero = jnp.zeros((16,), jnp.float32)

    acc = zero
    raws = [zero] * 5
    pltpu.sync_copy(pred_hbm.at[sample], chunk_v)
    for a in range(_NA):
        anw = float(_ANCHORS[2 * a])
        anh = float(_ANCHORS[2 * a + 1])
        besteq = best == float(a)

        def body(i, carry):
            acc, rx, ry, rw, rh, rc = carry
            k = i * 16 + lane
            kf = k.astype(jnp.float32)
            gyi = ((k * 40330) >> 21).astype(jnp.float32)
            gxi = kf - gyi * _W
            xr = chunk_v[5 * a + 0, pl.ds(i * 16, 16)]
            yr = chunk_v[5 * a + 1, pl.ds(i * 16, 16)]
            wr = chunk_v[5 * a + 2, pl.ds(i * 16, 16)]
            hr = chunk_v[5 * a + 3, pl.ds(i * 16, 16)]
            cr = chunk_v[5 * a + 4, pl.ds(i * 16, 16)]
            sigx = 1.0 / (1.0 + jnp.exp(-xr))
            sigy = 1.0 / (1.0 + jnp.exp(-yr))
            pcv = 1.0 / (1.0 + jnp.exp(-cr))
            pwv = jnp.exp(wr) * anw
            phv = jnp.exp(hr) * anh
            pxv = sigx + gxi
            pyv = sigy + gyi
            hw_ = 0.5 * pwv
            hh_ = 0.5 * phv
            uw = jnp.maximum(gx1, pxv + hw_) - jnp.minimum(gx0, pxv - hw_)
            uh = jnp.maximum(gy1, pyv + hh_) - jnp.minimum(gy0, pyv - hh_)
            cwv = gw + pwv - uw
            chv = gh + phv - uh
            carea = cwv * chv
            uarea = garea + pwv * phv - carea
            sil = (cwv > 0.0) & (chv > 0.0) & (carea > 0.6 * uarea)
            ddx = sigx - 0.5
            ddy = sigy - 0.5
            cell = ddx * ddx + ddy * ddy + wr * wr + hr * hr \
                + jnp.where(sil, 0.0, pcv * pcv)
            selm = besteq & (kf == kmatch)
            rx = rx + jnp.where(selm, xr, 0.0)
            ry = ry + jnp.where(selm, yr, 0.0)
            rw = rw + jnp.where(selm, wr, 0.0)
            rh = rh + jnp.where(selm, hr, 0.0)
            rc = rc + jnp.where(selm, cr, 0.0)
            return (acc + cell, rx, ry, rw, rh, rc)

        acc, raws[0], raws[1], raws[2], raws[3], raws[4] = lax.fori_loop(
            0, _HW // 16, body, (acc, raws[0], raws[1], raws[2], raws[3],
                                 raws[4]))

    def tot(v):
        for bit in (1, 2, 4, 8):
            idx = lane ^ bit
            v = v + v.at[idx].get(mode="promise_in_bounds")
        return v

    r0, r1, r2, r3, r4 = (tot(v) for v in raws)
    sxm = 1.0 / (1.0 + jnp.exp(-r0))
    sym = 1.0 / (1.0 + jnp.exp(-r1))
    pcm = 1.0 / (1.0 + jnp.exp(-r4))
    pwm = jnp.exp(r2) * awb
    phm = jnp.exp(r3) * ahb
    pxm = sxm + gif
    pym = sym + gjf
    uwm = jnp.maximum(gx1, pxm + 0.5 * pwm) - jnp.minimum(gx0, pxm - 0.5 * pwm)
    uhm = jnp.maximum(gy1, pym + 0.5 * phm) - jnp.minimum(gy0, pym - 0.5 * phm)
    cwm = gw + pwm - uwm
    chm = gh + phm - uhm
    cam = cwm * chm
    uam = garea + pwm * phm - cam
    iou_t = jnp.where((cwm > 0.0) & (chm > 0.0), cam / uam, 0.0)

    def sq(v):
        return v * v

    coord_corr = sq(sxm - tx) - sq(sxm - 0.5) \
        + sq(sym - ty) - sq(sym - 0.5) \
        + sq(r2 - tw) - r2 * r2 \
        + sq(r3 - th) - r3 * r3
    dconf = pcm - iou_t
    conf_corr = 5.0 * dconf * dconf \
        - jnp.where(iou_t > 0.6, 0.0, pcm * pcm)
    acc = acc + (coord_corr + conf_corr) * (1.0 / 16.0)

    acc_v[...] = acc
    pltpu.sync_copy(acc_v, out_hbm.at[pl.ds(wid * 16, 16)])


def kernel(pred, target):
    bs = pred.shape[0]
    pred3 = pred.reshape(bs, _NA * 5, _HW)
    params = pl.pallas_call(
        _prep_kernel,
        out_shape=jax.ShapeDtypeStruct((bs, 16), jnp.float32),
    )(target)

    tc_total = pl.pallas_call(
        _dense_kernel,
        grid=(_NTC // _G,),
        in_specs=[
            pl.BlockSpec((_G, 16), lambda g: (g, 0)),
            pl.BlockSpec((_G, _NA * 5, _HW), lambda g: (g, 0, 0)),
        ],
        out_specs=pl.BlockSpec((1, 1), lambda g: (0, 0)),
        out_shape=jax.ShapeDtypeStruct((1, 1), jnp.float32),
    )(params, pred3)

    mesh = plsc.VectorSubcoreMesh(core_axis_name="c", subcore_axis_name="s")
    sc_part = functools.partial(
        pl.kernel, mesh=mesh,
        out_type=jax.ShapeDtypeStruct((_NSC * 16,), jnp.float32),
        scratch_types=[
            pltpu.VMEM((_NA * 5, _HW), jnp.float32),
            pltpu.VMEM((256,), jnp.float32),
            pltpu.VMEM((16,), jnp.float32),
        ],
    )(_sc_kernel)
    params_sc = jnp.broadcast_to(
        params[:, :, None], (bs, 16, 16)).reshape(-1)
    sc_out = sc_part(pred3, params_sc)

    return (tc_total[0, 0] + jnp.sum(sc_out)) * 0.5
